# baseline (speedup 1.0000x reference)
import functools

import jax
import jax.numpy as jnp
from jax import lax
from jax.experimental import pallas as pl
from jax.experimental.pallas import tpu as pltpu
from jax.experimental.pallas import tpu_sc as plsc

_NC, _NS = 2, 16
_NW = _NC * _NS
_CHUNK = 512


@functools.lru_cache(maxsize=None)
def _make_gather(n_rows: int, d: int, chunk: int):
    per_w = n_rows // _NW
    assert per_w * _NW == n_rows and per_w % chunk == 0
    n_chunks = per_w // chunk
    mesh = plsc.VectorSubcoreMesh(
        core_axis_name="c", subcore_axis_name="s",
        num_cores=_NC, num_subcores=_NS)

    @functools.partial(
        pl.kernel,
        out_type=jax.ShapeDtypeStruct((n_rows, d), jnp.float32),
        mesh=mesh,
        scratch_types=[
            pltpu.VMEM((chunk,), jnp.int32),
            pltpu.VMEM((chunk, d), jnp.float32),
            pltpu.SemaphoreType.DMA,
        ],
        compiler_params=pltpu.CompilerParams(use_tc_tiling_on_sc=False),
    )
    def gather(idx_hbm, table_hbm, out_hbm, idx_v, rows_v, sem):
        wid = lax.axis_index("s") * _NC + lax.axis_index("c")
        base = wid * per_w

        @pl.loop(0, n_chunks)
        def _(i):
            off = base + i * chunk
            pltpu.sync_copy(idx_hbm.at[pl.ds(off, chunk)], idx_v)
            pltpu.async_copy(table_hbm.at[idx_v], rows_v, sem).wait()
            pltpu.sync_copy(rows_v, out_hbm.at[pl.ds(off, chunk)])

    return gather


def kernel(indices, table):
    b, l = indices.shape
    d = table.shape[1]
    idx = indices.reshape(b * l)
    out = _make_gather(b * l, d, _CHUNK)(idx, table)
    return out.reshape(b, l, d)


# speedup vs baseline: 1.0411x; 1.0411x over previous
import functools

import jax
import jax.numpy as jnp
from jax import lax
from jax.experimental import pallas as pl
from jax.experimental.pallas import tpu as pltpu
from jax.experimental.pallas import tpu_sc as plsc

_NC, _NS = 2, 16
_NW = _NC * _NS
_CHUNK = 512
_NBUF = 2


@functools.lru_cache(maxsize=None)
def _make_gather(n_rows: int, d: int, chunk: int):
    per_w = n_rows // _NW
    assert per_w * _NW == n_rows and per_w % chunk == 0
    n_chunks = per_w // chunk
    assert n_chunks % _NBUF == 0
    mesh = plsc.VectorSubcoreMesh(
        core_axis_name="c", subcore_axis_name="s",
        num_cores=_NC, num_subcores=_NS)

    @functools.partial(
        pl.kernel,
        out_type=jax.ShapeDtypeStruct((n_rows, d), jnp.float32),
        mesh=mesh,
        scratch_types=[
            pltpu.VMEM((_NBUF, chunk), jnp.int32),
            pltpu.VMEM((_NBUF, chunk, d), jnp.float32),
            [pltpu.SemaphoreType.DMA] * _NBUF,
            [pltpu.SemaphoreType.DMA] * _NBUF,
            [pltpu.SemaphoreType.DMA] * _NBUF,
        ],
        compiler_params=pltpu.CompilerParams(use_tc_tiling_on_sc=False),
    )
    def gather(idx_hbm, table_hbm, out_hbm, idx_v, rows_v, si, sg, so):
        wid = lax.axis_index("s") * _NC + lax.axis_index("c")
        base = wid * per_w

        def idx_slice(g):
            return idx_hbm.at[pl.ds(base + g * chunk, chunk)]

        def out_slice(g):
            return out_hbm.at[pl.ds(base + g * chunk, chunk)]

        def start_idx(g, b):
            pltpu.async_copy(idx_slice(g), idx_v.at[b], si[b])

        def wait_idx(b):
            pltpu.make_async_copy(idx_slice(0), idx_v.at[b], si[b]).wait()

        def start_gather(b):
            pltpu.async_copy(table_hbm.at[idx_v.at[b]], rows_v.at[b], sg[b])

        def wait_gather(b):
            pltpu.make_async_copy(
                table_hbm.at[idx_v.at[b]], rows_v.at[b], sg[b]).wait()

        def start_store(g, b):
            pltpu.async_copy(rows_v.at[b], out_slice(g), so[b])

        def wait_store(b):
            pltpu.make_async_copy(rows_v.at[b], out_slice(0), so[b]).wait()

        for b in range(_NBUF):
            start_idx(b, b)

        @pl.loop(0, n_chunks, step=_NBUF)
        def _(i):
            for b in range(_NBUF):
                g = i + b

                @pl.when(g >= _NBUF)
                def _():
                    wait_store(b)

                wait_idx(b)
                start_gather(b)
                wait_gather(b)

                @pl.when(g + _NBUF < n_chunks)
                def _():
                    start_idx(g + _NBUF, b)

                start_store(g, b)

        for b in range(_NBUF):
            wait_store(b)

    return gather


def kernel(indices, table):
    b, l = indices.shape
    d = table.shape[1]
    idx = indices.reshape(b * l)
    out = _make_gather(b * l, d, _CHUNK)(idx, table)
    return out.reshape(b, l, d)


# speedup vs baseline: 1.0414x; 1.0003x over previous
import functools

import jax
import jax.numpy as jnp
from jax import lax
from jax.experimental import pallas as pl
from jax.experimental.pallas import tpu as pltpu
from jax.experimental.pallas import tpu_sc as plsc

_NC, _NS = 2, 16
_NW = _NC * _NS
_NB = 4
_NBUF = 2


@functools.lru_cache(maxsize=None)
def _make_gather(bsz: int, seq: int, d: int):
    per_w = bsz // _NW
    assert per_w * _NW == bsz and per_w % _NB == 0
    n_chunks = per_w // _NB
    assert n_chunks % _NBUF == 0
    mesh = plsc.VectorSubcoreMesh(
        core_axis_name="c", subcore_axis_name="s",
        num_cores=_NC, num_subcores=_NS)

    @functools.partial(
        pl.kernel,
        out_type=jax.ShapeDtypeStruct((bsz, seq, d), jnp.float32),
        mesh=mesh,
        scratch_types=[
            pltpu.VMEM((_NBUF, _NB, seq), jnp.int32),
            pltpu.VMEM((_NBUF, _NB, seq, d), jnp.float32),
            [pltpu.SemaphoreType.DMA] * _NBUF,
            [pltpu.SemaphoreType.DMA] * _NBUF,
            [pltpu.SemaphoreType.DMA] * _NBUF,
        ],
        compiler_params=pltpu.CompilerParams(use_tc_tiling_on_sc=False),
    )
    def gather(idx_hbm, table_hbm, out_hbm, idx_v, rows_v, si, sg, so):
        wid = lax.axis_index("s") * _NC + lax.axis_index("c")
        base = wid * per_w

        def idx_slice(g):
            return idx_hbm.at[pl.ds(base + g * _NB, _NB)]

        def out_slice(g):
            return out_hbm.at[pl.ds(base + g * _NB, _NB)]

        def start_idx(g, b):
            pltpu.async_copy(idx_slice(g), idx_v.at[b], si[b])

        def wait_idx(b):
            pltpu.make_async_copy(idx_slice(0), idx_v.at[b], si[b]).wait()

        def start_gather(b):
            for j in range(_NB):
                pltpu.async_copy(
                    table_hbm.at[idx_v.at[b, j]], rows_v.at[b, j], sg[b])

        def wait_gather(b):
            for j in range(_NB):
                pltpu.make_async_copy(
                    table_hbm.at[idx_v.at[b, j]], rows_v.at[b, j],
                    sg[b]).wait()

        def start_store(g, b):
            pltpu.async_copy(rows_v.at[b], out_slice(g), so[b])

        def wait_store(b):
            pltpu.make_async_copy(rows_v.at[b], out_slice(0), so[b]).wait()

        for b in range(_NBUF):
            start_idx(b, b)

        @pl.loop(0, n_chunks, step=_NBUF)
        def _(i):
            for b in range(_NBUF):
                g = i + b

                @pl.when(g >= _NBUF)
                def _():
                    wait_store(b)

                wait_idx(b)
                start_gather(b)
                wait_gather(b)

                @pl.when(g + _NBUF < n_chunks)
                def _():
                    start_idx(g + _NBUF, b)

                start_store(g, b)

        for b in range(_NBUF):
            wait_store(b)

    return gather


def kernel(indices, table):
    bsz, seq = indices.shape
    d = table.shape[1]
    return _make_gather(bsz, seq, d)(indices, table)


# speedup vs baseline: 1.2722x; 1.2216x over previous
import functools

import jax
import jax.numpy as jnp
from jax import lax
from jax.experimental import pallas as pl
from jax.experimental.pallas import tpu as pltpu
from jax.experimental.pallas import tpu_sc as plsc

_NC, _NS = 2, 16
_NW = _NC * _NS
_CHUNK = 400
_NBUF = 2
_PD = 128


@functools.lru_cache(maxsize=None)
def _make_gather(n_rows: int, d: int, chunk: int):
    per_w = n_rows // _NW
    assert per_w * _NW == n_rows and per_w % chunk == 0
    n_chunks = per_w // chunk
    assert n_chunks % _NBUF == 0
    mesh = plsc.VectorSubcoreMesh(
        core_axis_name="c", subcore_axis_name="s",
        num_cores=_NC, num_subcores=_NS)

    @functools.partial(
        pl.kernel,
        out_type=jax.ShapeDtypeStruct((n_rows, _PD), jnp.float32),
        mesh=mesh,
        scratch_types=[
            pltpu.VMEM((_NBUF, chunk), jnp.int32),
            pltpu.VMEM((_NBUF, chunk, _PD), jnp.float32),
            [pltpu.SemaphoreType.DMA] * _NBUF,
            [pltpu.SemaphoreType.DMA] * _NBUF,
            [pltpu.SemaphoreType.DMA] * _NBUF,
        ],
        compiler_params=pltpu.CompilerParams(use_tc_tiling_on_sc=False),
    )
    def gather(idx_hbm, table2_hbm, out_hbm, idx_v, rows_v, si, sg, so):
        table_hbm = table2_hbm
        wid = lax.axis_index("s") * _NC + lax.axis_index("c")
        base = wid * per_w

        def idx_slice(g):
            return idx_hbm.at[pl.ds(base + g * chunk, chunk)]

        def out_slice(g):
            return out_hbm.at[pl.ds(base + g * chunk, chunk)]

        def start_idx(g, b):
            pltpu.async_copy(idx_slice(g), idx_v.at[b], si[b])

        def wait_idx(b):
            pltpu.make_async_copy(idx_slice(0), idx_v.at[b], si[b]).wait()

        def start_gather(b):
            pltpu.async_copy(
                table_hbm.at[idx_v.at[b]],
                rows_v.at[b], sg[b])

        def wait_gather(b):
            pltpu.make_async_copy(
                table_hbm.at[idx_v.at[b]],
                rows_v.at[b], sg[b]).wait()

        def start_store(g, b):
            pltpu.async_copy(rows_v.at[b], out_slice(g), so[b])

        def wait_store(b):
            pltpu.make_async_copy(rows_v.at[b], out_slice(0), so[b]).wait()

        for b in range(_NBUF):
            start_idx(b, b)

        @pl.loop(0, n_chunks, step=_NBUF)
        def _(i):
            for b in range(_NBUF):
                g = i + b

                @pl.when(g >= _NBUF)
                def _():
                    wait_store(b)

                wait_idx(b)
                start_gather(b)
                wait_gather(b)

                @pl.when(g + _NBUF < n_chunks)
                def _():
                    start_idx(g + _NBUF, b)

                start_store(g, b)

        for b in range(_NBUF):
            wait_store(b)

    return gather


def kernel(indices, table):
    bsz, seq = indices.shape
    v, d = table.shape
    idx = indices.reshape(bsz * seq)
    tpad = jnp.pad(table, ((0, 0), (0, _PD - d)))
    out = _make_gather(bsz * seq, d, _CHUNK)(idx, tpad)
    return out[:, :d].reshape(bsz, seq, d)


# speedup vs baseline: 1.3514x; 1.0623x over previous
import functools

import jax
import jax.numpy as jnp
from jax import lax
from jax.experimental import pallas as pl
from jax.experimental.pallas import tpu as pltpu
from jax.experimental.pallas import tpu_sc as plsc

_NC, _NS = 2, 16
_NW = _NC * _NS
_CHUNK = 400
_NBUF = 2
_PD = 128
_TBLK = 2048


@functools.lru_cache(maxsize=None)
def _make_transpose(v: int, d: int):
    grid = pl.cdiv(v, _TBLK)

    def body(x_ref, o_ref):
        xt = jnp.swapaxes(x_ref[...], 0, 1)
        o_ref[...] = jnp.pad(xt, ((0, 0), (0, _PD - d)))

    return pl.pallas_call(
        body,
        grid=(grid,),
        in_specs=[pl.BlockSpec((d, _TBLK), lambda i: (0, i))],
        out_specs=pl.BlockSpec((_TBLK, _PD), lambda i: (i, 0)),
        out_shape=jax.ShapeDtypeStruct((v, _PD), jnp.float32),
    )


@functools.lru_cache(maxsize=None)
def _make_gather(n_rows: int, chunk: int):
    per_w = n_rows // _NW
    assert per_w * _NW == n_rows and per_w % chunk == 0
    n_chunks = per_w // chunk
    assert n_chunks % _NBUF == 0
    mesh = plsc.VectorSubcoreMesh(
        core_axis_name="c", subcore_axis_name="s",
        num_cores=_NC, num_subcores=_NS)

    @functools.partial(
        pl.kernel,
        out_type=jax.ShapeDtypeStruct((n_rows, _PD), jnp.float32),
        mesh=mesh,
        scratch_types=[
            pltpu.VMEM((_NBUF, chunk), jnp.int32),
            pltpu.VMEM((_NBUF, chunk, _PD), jnp.float32),
            [pltpu.SemaphoreType.DMA] * _NBUF,
            [pltpu.SemaphoreType.DMA] * _NBUF,
            [pltpu.SemaphoreType.DMA] * _NBUF,
        ],
        compiler_params=pltpu.CompilerParams(use_tc_tiling_on_sc=False),
    )
    def gather(idx_hbm, table_hbm, out_hbm, idx_v, rows_v, si, sg, so):
        wid = lax.axis_index("s") * _NC + lax.axis_index("c")
        base = wid * per_w

        def idx_slice(g):
            return idx_hbm.at[pl.ds(base + g * chunk, chunk)]

        def out_slice(g):
            return out_hbm.at[pl.ds(base + g * chunk, chunk)]

        def start_idx(g, b):
            pltpu.async_copy(idx_slice(g), idx_v.at[b], si[b])

        def wait_idx(b):
            pltpu.make_async_copy(idx_slice(0), idx_v.at[b], si[b]).wait()

        def start_gather(b):
            pltpu.async_copy(table_hbm.at[idx_v.at[b]], rows_v.at[b], sg[b])

        def wait_gather(b):
            pltpu.make_async_copy(
                table_hbm.at[idx_v.at[b]], rows_v.at[b], sg[b]).wait()

        def start_store(g, b):
            pltpu.async_copy(rows_v.at[b], out_slice(g), so[b])

        def wait_store(b):
            pltpu.make_async_copy(rows_v.at[b], out_slice(0), so[b]).wait()

        for b in range(_NBUF):
            start_idx(b, b)

        @pl.loop(0, n_chunks, step=_NBUF)
        def _(i):
            for b in range(_NBUF):
                g = i + b

                @pl.when(g >= _NBUF)
                def _():
                    wait_store(b)

                wait_idx(b)
                start_gather(b)
                wait_gather(b)

                @pl.when(g + _NBUF < n_chunks)
                def _():
                    start_idx(g + _NBUF, b)

                start_store(g, b)

        for b in range(_NBUF):
            wait_store(b)

    return gather


def kernel(indices, table):
    bsz, seq = indices.shape
    v, d = table.shape
    idx = indices.reshape(bsz * seq)
    tpad = _make_transpose(v, d)(jnp.swapaxes(table, 0, 1))
    out = _make_gather(bsz * seq, _CHUNK)(idx, tpad)
    return out[:, :d].reshape(bsz, seq, d)


# speedup vs baseline: 2.1424x; 1.5853x over previous
import functools

import jax
import jax.numpy as jnp
from jax import lax
from jax.experimental import pallas as pl
from jax.experimental.pallas import tpu as pltpu
from jax.experimental.pallas import tpu_sc as plsc

_NC, _NS = 2, 16
_NW = _NC * _NS
_CHUNK = 800
_NBUF = 2
_PD = 128
_TBLK = 4096
_LANES = 16


@functools.lru_cache(maxsize=None)
def _make_transpose(v: int, d: int, split: int, n_lines: int):
    g1 = split // _TBLK
    grid = pl.cdiv(n_lines, _TBLK)

    def body(x1_ref, x2_ref, o_ref):
        o_ref[:, 0:d] = jnp.swapaxes(x1_ref[...], 0, 1)
        o_ref[:, d:2 * d] = jnp.swapaxes(x2_ref[...], 0, 1)

    return pl.pallas_call(
        body,
        grid=(grid,),
        in_specs=[
            pl.BlockSpec((d, _TBLK), lambda i: (0, i)),
            pl.BlockSpec((d, _TBLK), lambda i: (0, i + g1)),
        ],
        out_specs=pl.BlockSpec((_TBLK, 2 * d), lambda i: (i, 0)),
        out_shape=jax.ShapeDtypeStruct((n_lines, 2 * d), jnp.float32),
    )


@functools.lru_cache(maxsize=None)
def _make_gather(n_rows: int, d: int, chunk: int, split: int):
    per_w = n_rows // _NW
    assert per_w * _NW == n_rows and per_w % chunk == 0
    n_chunks = per_w // chunk
    assert n_chunks % _NBUF == 0 and chunk % _LANES == 0
    mesh = plsc.VectorSubcoreMesh(
        core_axis_name="c", subcore_axis_name="s",
        num_cores=_NC, num_subcores=_NS)

    @functools.partial(
        pl.kernel,
        out_type=jax.ShapeDtypeStruct((n_rows, _PD), jnp.float32),
        mesh=mesh,
        scratch_types=[
            pltpu.VMEM((_NBUF, chunk), jnp.int32),
            pltpu.VMEM((_NBUF, chunk, d), jnp.float32),
            [pltpu.SemaphoreType.DMA] * _NBUF,
            [pltpu.SemaphoreType.DMA] * _NBUF,
            [pltpu.SemaphoreType.DMA] * _NBUF,
        ],
        compiler_params=pltpu.CompilerParams(use_tc_tiling_on_sc=False),
    )
    def gather(idx_hbm, table_hbm, out_hbm, idx_v, rows_v, si, sg, so):
        wid = lax.axis_index("s") * _NC + lax.axis_index("c")
        base = wid * per_w

        def idx_slice(g):
            return idx_hbm.at[pl.ds(base + g * chunk, chunk)]

        def out_slice(g):
            return out_hbm.at[pl.ds(base + g * chunk, chunk), pl.ds(0, d)]

        def start_idx(g, b):
            pltpu.async_copy(idx_slice(g), idx_v.at[b], si[b])

        def wait_idx(b):
            pltpu.make_async_copy(idx_slice(0), idx_v.at[b], si[b]).wait()

        def remap_idx(b):
            @pl.loop(0, chunk, step=_LANES)
            def _(j):
                i = idx_v[b, pl.ds(j, _LANES)]
                r = jnp.where(i < split, i + i, i + i - (2 * split - 1))
                idx_v[b, pl.ds(j, _LANES)] = r

        def start_gather(b):
            pltpu.async_copy(table_hbm.at[idx_v.at[b]], rows_v.at[b], sg[b])

        def wait_gather(b):
            pltpu.make_async_copy(
                table_hbm.at[idx_v.at[b]], rows_v.at[b], sg[b]).wait()

        def start_store(g, b):
            pltpu.async_copy(rows_v.at[b], out_slice(g), so[b])

        def wait_store(b):
            pltpu.make_async_copy(rows_v.at[b], out_slice(0), so[b]).wait()

        for b in range(_NBUF):
            start_idx(b, b)

        @pl.loop(0, n_chunks, step=_NBUF)
        def _(i):
            for b in range(_NBUF):
                g = i + b

                @pl.when(g >= _NBUF)
                def _():
                    wait_store(b)

                wait_idx(b)
                remap_idx(b)
                start_gather(b)
                wait_gather(b)

                @pl.when(g + _NBUF < n_chunks)
                def _():
                    start_idx(g + _NBUF, b)

                start_store(g, b)

        for b in range(_NBUF):
            wait_store(b)

    return gather


def kernel(indices, table):
    bsz, seq = indices.shape
    v, d = table.shape
    split = _TBLK * (v // 2 // _TBLK)
    n_lines = v - split
    idx = indices.reshape(bsz * seq)
    t_t = jnp.swapaxes(table, 0, 1)
    t2 = _make_transpose(v, d, split, n_lines)(t_t, t_t)
    t_lin = t2.reshape(2 * n_lines, d)
    out = _make_gather(bsz * seq, d, _CHUNK, split)(idx, t_lin)
    return out[:, :d].reshape(bsz, seq, d)


# speedup vs baseline: 2.2993x; 1.0732x over previous
import functools

import jax
import jax.numpy as jnp
from jax import lax
from jax.experimental import pallas as pl
from jax.experimental.pallas import tpu as pltpu
from jax.experimental.pallas import tpu_sc as plsc

_NC, _NS = 2, 16
_NW = _NC * _NS
_CHUNK = 800
_NBUF = 2
_PD = 128
_TBLK = 16384
_TSUB = 512
_LANES = 16


@functools.lru_cache(maxsize=None)
def _make_transpose(v: int, d: int, split: int, n_lines: int):
    g1 = split // _TBLK
    grid = pl.cdiv(n_lines, _TBLK)

    def body(x1_ref, x2_ref, o_ref):
        for k in range(_TBLK // _TSUB):
            sl = pl.ds(k * _TSUB, _TSUB)
            o_ref[sl, 0:d] = jnp.swapaxes(x1_ref[:, sl], 0, 1)
            o_ref[sl, d:2 * d] = jnp.swapaxes(x2_ref[:, sl], 0, 1)

    return pl.pallas_call(
        body,
        grid=(grid,),
        in_specs=[
            pl.BlockSpec((d, _TBLK), lambda i: (0, i)),
            pl.BlockSpec((d, _TBLK), lambda i: (0, i + g1)),
        ],
        out_specs=pl.BlockSpec((_TBLK, 2 * d), lambda i: (i, 0)),
        out_shape=jax.ShapeDtypeStruct((n_lines, 2 * d), jnp.float32),
    )


@functools.lru_cache(maxsize=None)
def _make_gather(n_rows: int, d: int, chunk: int, split: int):
    per_w = n_rows // _NW
    assert per_w * _NW == n_rows and per_w % chunk == 0
    n_chunks = per_w // chunk
    assert n_chunks % _NBUF == 0 and chunk % _LANES == 0
    mesh = plsc.VectorSubcoreMesh(
        core_axis_name="c", subcore_axis_name="s",
        num_cores=_NC, num_subcores=_NS)

    @functools.partial(
        pl.kernel,
        out_type=jax.ShapeDtypeStruct((n_rows, _PD), jnp.float32),
        mesh=mesh,
        scratch_types=[
            pltpu.VMEM((_NBUF, chunk), jnp.int32),
            pltpu.VMEM((_NBUF, chunk, d), jnp.float32),
            [pltpu.SemaphoreType.DMA] * _NBUF,
            [pltpu.SemaphoreType.DMA] * _NBUF,
            [pltpu.SemaphoreType.DMA] * _NBUF,
        ],
        compiler_params=pltpu.CompilerParams(use_tc_tiling_on_sc=False),
    )
    def gather(idx_hbm, table_hbm, out_hbm, idx_v, rows_v, si, sg, so):
        wid = lax.axis_index("s") * _NC + lax.axis_index("c")
        base = wid * per_w

        def idx_slice(g):
            return idx_hbm.at[pl.ds(base + g * chunk, chunk)]

        def out_slice(g):
            return out_hbm.at[pl.ds(base + g * chunk, chunk), pl.ds(0, d)]

        def start_idx(g, b):
            pltpu.async_copy(idx_slice(g), idx_v.at[b], si[b])

        def wait_idx(b):
            pltpu.make_async_copy(idx_slice(0), idx_v.at[b], si[b]).wait()

        def remap_idx(b):
            @pl.loop(0, chunk, step=_LANES)
            def _(j):
                i = idx_v[b, pl.ds(j, _LANES)]
                r = jnp.where(i < split, i + i, i + i - (2 * split - 1))
                idx_v[b, pl.ds(j, _LANES)] = r

        def start_gather(b):
            pltpu.async_copy(table_hbm.at[idx_v.at[b]], rows_v.at[b], sg[b])

        def wait_gather(b):
            pltpu.make_async_copy(
                table_hbm.at[idx_v.at[b]], rows_v.at[b], sg[b]).wait()

        def start_store(g, b):
            pltpu.async_copy(rows_v.at[b], out_slice(g), so[b])

        def wait_store(b):
            pltpu.make_async_copy(rows_v.at[b], out_slice(0), so[b]).wait()

        for b in range(_NBUF):
            start_idx(b, b)

        @pl.loop(0, n_chunks, step=_NBUF)
        def _(i):
            for b in range(_NBUF):
                g = i + b

                @pl.when(g >= _NBUF)
                def _():
                    wait_store(b)

                wait_idx(b)
                remap_idx(b)
                start_gather(b)
                wait_gather(b)

                @pl.when(g + _NBUF < n_chunks)
                def _():
                    start_idx(g + _NBUF, b)

                start_store(g, b)

        for b in range(_NBUF):
            wait_store(b)

    return gather


def kernel(indices, table):
    bsz, seq = indices.shape
    v, d = table.shape
    split = _TBLK * (v // 2 // _TBLK)
    n_lines = v - split
    idx = indices.reshape(bsz * seq)
    t_t = jnp.swapaxes(table, 0, 1)
    t2 = _make_transpose(v, d, split, n_lines)(t_t, t_t)
    t_lin = t2.reshape(2 * n_lines, d)
    out = _make_gather(bsz * seq, d, _CHUNK, split)(idx, t_lin)
    return out[:, :d].reshape(bsz, seq, d)
